# baseline (speedup 1.0000x reference)
import functools

import jax
import jax.numpy as jnp
from jax import lax
from jax.experimental import pallas as pl
from jax.experimental.pallas import tpu as pltpu
from jax.experimental.pallas import tpu_sc as plsc

T, E, D, X = 4096, 8, 2048, 8
L = 16
NC, NS = 2, 16
NW = NC * NS
TS = 1280
TPW = TS // NW
PAIR = 2
GROUP = 8
BT = 256


def _wd_body(w_ref, idx_ref, wd_ref):
    w = w_ref[...]
    idx = idx_ref[...]
    cols = [jnp.sum(w * (idx == e).astype(jnp.float32), axis=1, keepdims=True)
            for e in range(E)]
    wd_ref[...] = jnp.concatenate(cols, axis=1)


def _densify_weights(weights_TX, indices_TX):
    return pl.pallas_call(
        _wd_body,
        out_shape=jax.ShapeDtypeStruct((T, E), jnp.float32),
    )(weights_TX, indices_TX)


def _sc_combine(dp2, wd_flat):
    mesh = plsc.VectorSubcoreMesh(core_axis_name="c", subcore_axis_name="s")

    @functools.partial(
        pl.kernel, mesh=mesh,
        compiler_params=pltpu.CompilerParams(needs_layout_passes=False),
        out_type=jax.ShapeDtypeStruct((TS, D), jnp.float32),
        scratch_types=[
            pltpu.VMEM((PAIR * E, D), jnp.float32),
            pltpu.VMEM((GROUP, D), jnp.float32),
            pltpu.VMEM((TPW * E,), jnp.float32),
        ],
    )
    def sc_kernel(dp_hbm, wd_hbm, out_hbm, slab_v, out_v, wd_v):
        wid = lax.axis_index("s") * NC + lax.axis_index("c")
        t_base = wid * TPW
        pltpu.sync_copy(wd_hbm.at[pl.ds(t_base * E, TPW * E)], wd_v)
        zeros16 = jnp.zeros((L,), jnp.int32)

        def group_body(g, carry):
            tg = t_base + g * GROUP
            for q in range(GROUP // PAIR):
                t0 = tg + q * PAIR
                pltpu.sync_copy(dp_hbm.at[pl.ds(t0 * E, PAIR * E)], slab_v)
                kbase = g * (GROUP * E) + q * (PAIR * E)
                wbc = [plsc.load_gather(wd_v, [zeros16 + (kbase + k)])
                       for k in range(PAIR * E)]

                def d_body(c, inner, q=q, wbc=wbc):
                    b = c * L
                    for tok in range(PAIR):
                        acc = slab_v[tok * E, pl.ds(b, L)] * wbc[tok * E]
                        for e in range(1, E):
                            acc = acc + slab_v[tok * E + e, pl.ds(b, L)] * wbc[tok * E + e]
                        out_v[q * PAIR + tok, pl.ds(b, L)] = acc
                    return inner

                lax.fori_loop(0, D // L, d_body, 0)
            pltpu.sync_copy(out_v, out_hbm.at[pl.ds(tg, GROUP)])
            return carry

        lax.fori_loop(0, TPW // GROUP, group_body, 0)

    return sc_kernel(dp2, wd_flat)


def _tc_combine_body(dp_ref, wd_ref, out_ref):
    dp = dp_ref[...]
    wd3 = wd_ref[...]
    out_ref[...] = (dp * wd3).sum(axis=1)


def _tc_combine_tail(down_proj_TED, wd_e1):
    off = TS // BT
    grid = ((T - TS) // BT,)
    return pl.pallas_call(
        _tc_combine_body,
        grid=grid,
        in_specs=[
            pl.BlockSpec((BT, E, D), lambda i: (i + off, 0, 0)),
            pl.BlockSpec((BT, E, 1), lambda i: (i + off, 0, 0)),
        ],
        out_specs=pl.BlockSpec((BT, D), lambda i: (i, 0)),
        out_shape=jax.ShapeDtypeStruct((T - TS, D), jnp.float32),
    )(down_proj_TED, wd_e1)


@jax.jit
def kernel(down_proj_TED, weights_TX, indices_TX):
    dp2 = down_proj_TED.reshape(T * E, D)
    wd = _densify_weights(weights_TX, indices_TX.astype(jnp.int32))
    out_sc = _sc_combine(dp2, wd.reshape(T * E))
    out_tc = _tc_combine_tail(down_proj_TED, wd[:, :, None])
    return jnp.concatenate([out_sc, out_tc], axis=0)


# speedup vs baseline: 1.1714x; 1.1714x over previous
import functools

import jax
import jax.numpy as jnp
from jax import lax
from jax.experimental import pallas as pl
from jax.experimental.pallas import tpu as pltpu
from jax.experimental.pallas import tpu_sc as plsc

T, E, D, X = 4096, 8, 2048, 8
L = 16
NC, NS = 2, 16
NW = NC * NS
TPW = T // NW
NPAIR = TPW // 2
BT = 256


def _sc_densify(w_flat, idx_flat):
    mesh = plsc.VectorSubcoreMesh(core_axis_name="c", subcore_axis_name="s")

    @functools.partial(
        pl.kernel, mesh=mesh,
        compiler_params=pltpu.CompilerParams(needs_layout_passes=False),
        out_type=jax.ShapeDtypeStruct((T * E,), jnp.float32),
        scratch_types=[
            pltpu.VMEM((TPW * X,), jnp.float32),
            pltpu.VMEM((TPW * X,), jnp.int32),
            pltpu.VMEM((TPW * E,), jnp.float32),
        ],
    )
    def sc_kernel(w_hbm, idx_hbm, wd_hbm, w_v, idx_v, wd_v):
        wid = lax.axis_index("s") * NC + lax.axis_index("c")
        base = wid * TPW * X
        pltpu.sync_copy(w_hbm.at[pl.ds(base, TPW * X)], w_v)
        pltpu.sync_copy(idx_hbm.at[pl.ds(base, TPW * X)], idx_v)
        iota = lax.broadcasted_iota(jnp.int32, (L,), 0)
        tok_off = iota & 8
        e_pat = iota & 7

        def pair_body(p, carry):
            pbase = p * (2 * X)
            wd16 = jnp.zeros((L,), jnp.float32)
            for x in range(X):
                gidx = tok_off + (pbase + x)
                wx = plsc.load_gather(w_v, [gidx])
                ix = plsc.load_gather(idx_v, [gidx])
                wd16 = wd16 + jnp.where(ix == e_pat, wx, jnp.float32(0.0))
            wd_v[pl.ds(p * (2 * E), 2 * E)] = wd16
            return carry

        lax.fori_loop(0, NPAIR, pair_body, 0)
        pltpu.sync_copy(wd_v, wd_hbm.at[pl.ds(wid * TPW * E, TPW * E)])

    return sc_kernel(w_flat, idx_flat)


def _tc_combine_body(dp_ref, wd_ref, out_ref):
    dp = dp_ref[...]
    wd3 = wd_ref[...]
    out_ref[...] = (dp * wd3).sum(axis=1)


def _tc_combine(down_proj_TED, wd_e1):
    grid = (T // BT,)
    return pl.pallas_call(
        _tc_combine_body,
        grid=grid,
        in_specs=[
            pl.BlockSpec((BT, E, D), lambda i: (i, 0, 0)),
            pl.BlockSpec((BT, E, 1), lambda i: (i, 0, 0)),
        ],
        out_specs=pl.BlockSpec((BT, D), lambda i: (i, 0)),
        out_shape=jax.ShapeDtypeStruct((T, D), jnp.float32),
    )(down_proj_TED, wd_e1)


@jax.jit
def kernel(down_proj_TED, weights_TX, indices_TX):
    w_flat = weights_TX.reshape(T * X)
    idx_flat = indices_TX.astype(jnp.int32).reshape(T * X)
    wd_flat = _sc_densify(w_flat, idx_flat)
    wd_e1 = wd_flat.reshape(T, E)[:, :, None]
    return _tc_combine(down_proj_TED, wd_e1)
